# baseline (speedup 1.0000x reference)
import functools

import jax
import jax.numpy as jnp
from jax import lax
from jax.experimental import pallas as pl
from jax.experimental.pallas import tpu as pltpu
from jax.experimental.pallas import tpu_sc as plsc

_B = 4096
_T = 50
_D = 64
_N = _B * _T
_NE = 1000000
_NC = 2
_NS = 16
_NW = _NC * _NS
_PER_W = _N // _NW
_NBUF = 2

_TXB = 512
_TXG = -(-_NE // _TXB)


def _tx_body(in_ref, out_ref):
    out_ref[:, 0:_D] = in_ref[...].T


@jax.jit
def _stage_ent(entT):
    return pl.pallas_call(
        _tx_body,
        grid=(_TXG,),
        in_specs=[pl.BlockSpec((_D, _TXB), lambda i: (0, i))],
        out_specs=pl.BlockSpec((_TXB, 128), lambda i: (i, 0)),
        out_shape=jax.ShapeDtypeStruct((_NE, 128), jnp.float32),
    )(entT)


def _gather_body(row_w, chunk, idx_hbm, table, out_hbm,
                 idx_v, buf0, buf1, g0, g1, w0, w1):
    nch = _PER_W // chunk
    wid = lax.axis_index("s") * _NC + lax.axis_index("c")
    base = wid * _PER_W
    bufs = (buf0, buf1)
    gsems = (g0, g1)
    wsems = (w0, w1)

    pltpu.sync_copy(idx_hbm.at[pl.ds(base, _PER_W)], idx_v)

    gdesc = [None] * _NBUF
    wdesc = [None] * _NBUF
    for c in range(nch):
        b = c % _NBUF
        if wdesc[b] is not None:
            wdesc[b].wait()
        gdesc[b] = pltpu.async_copy(
            table.at[idx_v.at[pl.ds(c * chunk, chunk)]], bufs[b], gsems[b])
        if c > 0:
            pb = (c - 1) % _NBUF
            gdesc[pb].wait()
            wdesc[pb] = pltpu.async_copy(
                bufs[pb].at[:, pl.ds(0, _D)] if row_w != _D else bufs[pb],
                out_hbm.at[pl.ds(base + (c - 1) * chunk, chunk)],
                wsems[pb])
    lb = (nch - 1) % _NBUF
    gdesc[lb].wait()
    wdesc[lb] = pltpu.async_copy(
        bufs[lb].at[:, pl.ds(0, _D)] if row_w != _D else bufs[lb],
        out_hbm.at[pl.ds(base + (nch - 1) * chunk, chunk)],
        wsems[lb])
    for d in wdesc:
        if d is not None:
            d.wait()


def _make_gather(row_w, chunk):
    mesh = plsc.VectorSubcoreMesh(
        core_axis_name="c", subcore_axis_name="s",
        num_cores=_NC, num_subcores=_NS)
    return pl.kernel(
        functools.partial(_gather_body, row_w, chunk),
        out_type=jax.ShapeDtypeStruct((_N, _D), jnp.float32),
        mesh=mesh,
        scratch_types=[
            pltpu.VMEM((_PER_W,), jnp.int32),
            pltpu.VMEM((chunk, row_w), jnp.float32),
            pltpu.VMEM((chunk, row_w), jnp.float32),
            pltpu.SemaphoreType.DMA,
            pltpu.SemaphoreType.DMA,
            pltpu.SemaphoreType.DMA,
            pltpu.SemaphoreType.DMA,
        ],
        compiler_params=pltpu.CompilerParams(use_tc_tiling_on_sc=False),
    )


@jax.jit
def _run(h_flat, r_flat, t_flat, ent, rel):
    rel2 = lax.optimization_barrier(rel.reshape(-1)).reshape(rel.shape)
    gather64 = _make_gather(_D, 800)
    orr = gather64(r_flat, rel2)
    ent128 = _stage_ent(ent.T)
    gather128 = _make_gather(128, 400)
    oh = gather128(h_flat, ent128)
    ot = gather128(t_flat, ent128)
    return oh, orr, ot


def kernel(h_id, r_id, t_id, ent_transfer, rel_transfer):
    h_flat = h_id.reshape(-1).astype(jnp.int32)
    r_flat = r_id.reshape(-1).astype(jnp.int32)
    t_flat = t_id.reshape(-1).astype(jnp.int32)
    oh, orr, ot = _run(h_flat, r_flat, t_flat,
                       ent_transfer, rel_transfer)
    shp = h_id.shape + (_D,)
    return (oh.reshape(shp), orr.reshape(shp), ot.reshape(shp))


# speedup vs baseline: 1.7331x; 1.7331x over previous
import functools

import jax
import jax.numpy as jnp
from jax import lax
from jax.experimental import pallas as pl
from jax.experimental.pallas import tpu as pltpu
from jax.experimental.pallas import tpu_sc as plsc

_B = 4096
_T = 50
_D = 64
_N = _B * _T
_NE = 1000000
_NC = 2
_NS = 16
_NW = _NC * _NS
_PER_W = _N // _NW
_NBUF = 2

_TXB = 2048
_TXG = -(-_NE // _TXB)


def _tx_body(in_ref, eye_ref, out_ref):
    out_ref[:, 0:_D] = jax.lax.dot_general(
        in_ref[...], eye_ref[...], (((0,), (0,)), ((), ())),
        preferred_element_type=jnp.float32)


@jax.jit
def _stage_ent(entT):
    eye = jnp.eye(_D, dtype=jnp.float32)
    return pl.pallas_call(
        _tx_body,
        grid=(_TXG,),
        in_specs=[pl.BlockSpec((_D, _TXB), lambda i: (0, i)),
                  pl.BlockSpec((_D, _D), lambda i: (0, 0))],
        out_specs=pl.BlockSpec((_TXB, 128), lambda i: (i, 0)),
        out_shape=jax.ShapeDtypeStruct((_NE, 128), jnp.float32),
    )(entT, eye)


def _gather_body(row_w, chunk, idx_hbm, table, out_hbm,
                 idx_v, buf0, buf1, g0, g1, w0, w1):
    nch = _PER_W // chunk
    wid = lax.axis_index("s") * _NC + lax.axis_index("c")
    base = wid * _PER_W
    bufs = (buf0, buf1)
    gsems = (g0, g1)
    wsems = (w0, w1)

    pltpu.sync_copy(idx_hbm.at[pl.ds(base, _PER_W)], idx_v)

    gdesc = [None] * _NBUF
    wdesc = [None] * _NBUF
    for c in range(nch):
        b = c % _NBUF
        if wdesc[b] is not None:
            wdesc[b].wait()
        gdesc[b] = pltpu.async_copy(
            table.at[idx_v.at[pl.ds(c * chunk, chunk)]], bufs[b], gsems[b])
        if c > 0:
            pb = (c - 1) % _NBUF
            gdesc[pb].wait()
            wdesc[pb] = pltpu.async_copy(
                bufs[pb].at[:, pl.ds(0, _D)] if row_w != _D else bufs[pb],
                out_hbm.at[pl.ds(base + (c - 1) * chunk, chunk)],
                wsems[pb])
    lb = (nch - 1) % _NBUF
    gdesc[lb].wait()
    wdesc[lb] = pltpu.async_copy(
        bufs[lb].at[:, pl.ds(0, _D)] if row_w != _D else bufs[lb],
        out_hbm.at[pl.ds(base + (nch - 1) * chunk, chunk)],
        wsems[lb])
    for d in wdesc:
        if d is not None:
            d.wait()


def _make_gather(row_w, chunk):
    mesh = plsc.VectorSubcoreMesh(
        core_axis_name="c", subcore_axis_name="s",
        num_cores=_NC, num_subcores=_NS)
    return pl.kernel(
        functools.partial(_gather_body, row_w, chunk),
        out_type=jax.ShapeDtypeStruct((_N, _D), jnp.float32),
        mesh=mesh,
        scratch_types=[
            pltpu.VMEM((_PER_W,), jnp.int32),
            pltpu.VMEM((chunk, row_w), jnp.float32),
            pltpu.VMEM((chunk, row_w), jnp.float32),
            pltpu.SemaphoreType.DMA,
            pltpu.SemaphoreType.DMA,
            pltpu.SemaphoreType.DMA,
            pltpu.SemaphoreType.DMA,
        ],
        compiler_params=pltpu.CompilerParams(use_tc_tiling_on_sc=False),
    )


@jax.jit
def _run(h_flat, r_flat, t_flat, ent, rel):
    rel2 = lax.optimization_barrier(rel.reshape(-1)).reshape(rel.shape)
    gather64 = _make_gather(_D, 800)
    orr = gather64(r_flat, rel2)
    ent128 = _stage_ent(ent.T)
    gather128 = _make_gather(128, 400)
    oh = gather128(h_flat, ent128)
    ot = gather128(t_flat, ent128)
    return oh, orr, ot


def kernel(h_id, r_id, t_id, ent_transfer, rel_transfer):
    h_flat = h_id.reshape(-1).astype(jnp.int32)
    r_flat = r_id.reshape(-1).astype(jnp.int32)
    t_flat = t_id.reshape(-1).astype(jnp.int32)
    oh, orr, ot = _run(h_flat, r_flat, t_flat,
                       ent_transfer, rel_transfer)
    shp = h_id.shape + (_D,)
    return (oh.reshape(shp), orr.reshape(shp), ot.reshape(shp))


# speedup vs baseline: 2.1918x; 1.2647x over previous
import functools

import jax
import jax.numpy as jnp
from jax import lax
from jax.experimental import pallas as pl
from jax.experimental.pallas import tpu as pltpu
from jax.experimental.pallas import tpu_sc as plsc

_B = 4096
_T = 50
_D = 64
_N = _B * _T
_NE = 1000000
_NC = 2
_NS = 16
_NW = _NC * _NS
_PER_W = _N // _NW
_NBUF = 2

_TXB = 8192
_TXG = -(-_NE // _TXB)


def _tx_body(in_ref, eye_ref, out_ref):
    out_ref[:, 0:_D] = jax.lax.dot_general(
        in_ref[...], eye_ref[...], (((0,), (0,)), ((), ())),
        preferred_element_type=jnp.float32)


@jax.jit
def _stage_ent(entT):
    eye = jnp.eye(_D, dtype=jnp.float32)
    return pl.pallas_call(
        _tx_body,
        grid=(_TXG,),
        in_specs=[pl.BlockSpec((_D, _TXB), lambda i: (0, i)),
                  pl.BlockSpec((_D, _D), lambda i: (0, 0))],
        out_specs=pl.BlockSpec((_TXB, 128), lambda i: (i, 0)),
        out_shape=jax.ShapeDtypeStruct((_NE, 128), jnp.float32),
    )(entT, eye)


def _gather_body(row_w, chunk, idx_hbm, table, out_hbm,
                 idx_v, buf0, buf1, g0, g1, w0, w1):
    nch = _PER_W // chunk
    wid = lax.axis_index("s") * _NC + lax.axis_index("c")
    base = wid * _PER_W
    bufs = (buf0, buf1)
    gsems = (g0, g1)
    wsems = (w0, w1)

    pltpu.sync_copy(idx_hbm.at[pl.ds(base, _PER_W)], idx_v)

    gdesc = [None] * _NBUF
    wdesc = [None] * _NBUF
    for c in range(nch):
        b = c % _NBUF
        if wdesc[b] is not None:
            wdesc[b].wait()
        gdesc[b] = pltpu.async_copy(
            table.at[idx_v.at[pl.ds(c * chunk, chunk)]], bufs[b], gsems[b])
        if c > 0:
            pb = (c - 1) % _NBUF
            gdesc[pb].wait()
            wdesc[pb] = pltpu.async_copy(
                bufs[pb].at[:, pl.ds(0, _D)] if row_w != _D else bufs[pb],
                out_hbm.at[pl.ds(base + (c - 1) * chunk, chunk)],
                wsems[pb])
    lb = (nch - 1) % _NBUF
    gdesc[lb].wait()
    wdesc[lb] = pltpu.async_copy(
        bufs[lb].at[:, pl.ds(0, _D)] if row_w != _D else bufs[lb],
        out_hbm.at[pl.ds(base + (nch - 1) * chunk, chunk)],
        wsems[lb])
    for d in wdesc:
        if d is not None:
            d.wait()


def _make_gather(row_w, chunk):
    mesh = plsc.VectorSubcoreMesh(
        core_axis_name="c", subcore_axis_name="s",
        num_cores=_NC, num_subcores=_NS)
    return pl.kernel(
        functools.partial(_gather_body, row_w, chunk),
        out_type=jax.ShapeDtypeStruct((_N, _D), jnp.float32),
        mesh=mesh,
        scratch_types=[
            pltpu.VMEM((_PER_W,), jnp.int32),
            pltpu.VMEM((chunk, row_w), jnp.float32),
            pltpu.VMEM((chunk, row_w), jnp.float32),
            pltpu.SemaphoreType.DMA,
            pltpu.SemaphoreType.DMA,
            pltpu.SemaphoreType.DMA,
            pltpu.SemaphoreType.DMA,
        ],
        compiler_params=pltpu.CompilerParams(use_tc_tiling_on_sc=False),
    )


@jax.jit
def _run(h_flat, r_flat, t_flat, ent, rel):
    rel2 = lax.optimization_barrier(rel.reshape(-1)).reshape(rel.shape)
    gather64 = _make_gather(_D, 800)
    orr = gather64(r_flat, rel2)
    ent128 = _stage_ent(ent.T)
    gather128 = _make_gather(128, 400)
    oh = gather128(h_flat, ent128)
    ot = gather128(t_flat, ent128)
    return oh, orr, ot


def kernel(h_id, r_id, t_id, ent_transfer, rel_transfer):
    h_flat = h_id.reshape(-1).astype(jnp.int32)
    r_flat = r_id.reshape(-1).astype(jnp.int32)
    t_flat = t_id.reshape(-1).astype(jnp.int32)
    oh, orr, ot = _run(h_flat, r_flat, t_flat,
                       ent_transfer, rel_transfer)
    shp = h_id.shape + (_D,)
    return (oh.reshape(shp), orr.reshape(shp), ot.reshape(shp))
